# baseline (speedup 1.0000x reference)
import dataclasses
import functools

import jax
import jax.numpy as jnp
from jax import lax
from jax.experimental import pallas as pl
from jax.experimental.pallas import tpu as pltpu
from jax.experimental.pallas import tpu_sc as plsc

N_NODES = 10000
N_EDGES = 320000
D = 128

NC = 2
NS = 16
L = 16

CHUNK = 128
CPW0 = 40
CPW1 = 120
SPC = 40
MAX_STAGES = max(CPW0, CPW1) // SPC
N_CHUNKS = NS * (CPW0 + CPW1)
E_PAD = N_CHUNKS * CHUNK
RBLK = 80
N_RBLK = N_NODES // RBLK
RB_T = (N_RBLK + NS - 1) // NS


def _matmul_body(x_ref, w_ref, o_ref):
    o_ref[...] = jnp.dot(x_ref[...], w_ref[...],
                         preferred_element_type=jnp.float32)


def _combine_body(p_ref, b_ref, o_ref):
    o_ref[...] = p_ref[0] + p_ref[1] + b_ref[...]


def _sc_spmm(h, row2, col, val):
    mesh = plsc.VectorSubcoreMesh(core_axis_name="c", subcore_axis_name="s")
    cp = pltpu.CompilerParams()
    if "needs_layout_passes" in pltpu.CompilerParams.__dataclass_fields__:
        cp = dataclasses.replace(cp, needs_layout_passes=False)

    @functools.partial(
        pl.kernel,
        compiler_params=cp,
        out_type=jax.ShapeDtypeStruct((NC, N_NODES, D), jnp.float32),
        mesh=mesh,
        scratch_types=[
            pltpu.VMEM((SPC * CHUNK,), jnp.int32),
            pltpu.VMEM((SPC, CHUNK), jnp.int32),
            pltpu.VMEM((SPC * CHUNK,), jnp.float32),
            pltpu.VMEM((CHUNK, D), jnp.float32),
            pltpu.VMEM((CHUNK, D), jnp.float32),
            pltpu.VMEM_SHARED((N_NODES, D), jnp.float32),
            pltpu.SemaphoreType.DMA,
            pltpu.SemaphoreType.DMA,
        ],
    )
    def spmm_kernel(h_hbm, row_hbm, col_hbm, val_hbm, out_hbm,
                    col_v, row_v, val_v, rows_a, rows_b, acc_sh,
                    sem_a, sem_b):
        cid = lax.axis_index("c")
        sid = lax.axis_index("s")

        @pl.loop(0, RBLK)
        def _(e):
            for k in range(D // L):
                rows_a[e, pl.ds(k * L, L)] = jnp.zeros((L,), jnp.float32)

        @pl.loop(0, RB_T)
        def _(t):
            blk = sid + t * NS

            @pl.when(blk < N_RBLK)
            def _():
                pltpu.sync_copy(rows_a.at[pl.ds(0, RBLK)],
                                acc_sh.at[pl.ds(blk * RBLK, RBLK)])

        plsc.subcore_barrier()

        def start_gather(j, buf, sem):
            pltpu.async_copy(
                h_hbm.at[col_v.at[pl.ds(j * CHUNK, CHUNK)]], buf, sem)

        def wait_gather(j, buf, sem):
            pltpu.make_async_copy(
                h_hbm.at[col_v.at[pl.ds(j * CHUNK, CHUNK)]], buf, sem).wait()

        def scale(buf, j):
            @pl.loop(0, CHUNK // L)
            def _(g):
                base_e = j * CHUNK + g * L
                for e in range(L):
                    bcast = plsc.load_gather(
                        val_v, [jnp.full((L,), base_e + e, jnp.int32)])
                    r = g * L + e
                    for k in range(D // L):
                        sl = pl.ds(k * L, L)
                        buf[r, sl] = buf[r, sl] * bcast

        def scatter(buf, j):
            pltpu.sync_copy(buf, acc_sh.at[row_v.at[j]], add=True)

        cpw = jnp.where(cid == 0, CPW0, CPW1)
        cbase = cid * NS * CPW0 + sid * cpw

        for s in range(MAX_STAGES):

            @pl.when(s * SPC < cpw)
            def _():
                stage_c = cbase + s * SPC
                ebase = stage_c * CHUNK
                pltpu.sync_copy(col_hbm.at[pl.ds(ebase, SPC * CHUNK)], col_v)
                pltpu.sync_copy(val_hbm.at[pl.ds(ebase, SPC * CHUNK)], val_v)
                pltpu.sync_copy(row_hbm.at[pl.ds(stage_c, SPC)], row_v)

                start_gather(0, rows_a, sem_a)

                @pl.loop(0, SPC // 2)
                def _(t):
                    j0 = t * 2
                    start_gather(j0 + 1, rows_b, sem_b)

                    wait_gather(j0, rows_a, sem_a)
                    scale(rows_a, j0)
                    scatter(rows_a, j0)

                    @pl.when(j0 + 2 < SPC)
                    def _():
                        start_gather(j0 + 2, rows_a, sem_a)

                    wait_gather(j0 + 1, rows_b, sem_b)
                    scale(rows_b, j0 + 1)
                    scatter(rows_b, j0 + 1)

        plsc.subcore_barrier()

        @pl.loop(0, RB_T)
        def _(t):
            blk = sid + t * NS

            @pl.when(blk < N_RBLK)
            def _():
                pltpu.sync_copy(
                    acc_sh.at[pl.ds(blk * RBLK, RBLK)],
                    out_hbm.at[cid, pl.ds(blk * RBLK, RBLK)])

    return spmm_kernel(h, row2, col, val)


def kernel(x, edge_index, adj_values, W, b):
    row = edge_index[0].astype(jnp.int32)
    col = edge_index[1].astype(jnp.int32)
    val = adj_values.astype(jnp.float32)

    pad = E_PAD - N_EDGES
    row2 = jnp.pad(row, (0, pad)).reshape(N_CHUNKS, CHUNK)
    col = jnp.pad(col, (0, pad))
    val = jnp.pad(val, (0, pad))

    h = pl.pallas_call(
        _matmul_body,
        grid=(10,),
        in_specs=[
            pl.BlockSpec((N_NODES // 10, D), lambda i: (i, 0)),
            pl.BlockSpec((D, D), lambda i: (0, 0)),
        ],
        out_specs=pl.BlockSpec((N_NODES // 10, D), lambda i: (i, 0)),
        out_shape=jax.ShapeDtypeStruct((N_NODES, D), jnp.float32),
    )(x, W)

    partials = _sc_spmm(h, row2, col, val)

    b2 = b.reshape(1, D).astype(jnp.float32)
    out = pl.pallas_call(
        _combine_body,
        grid=(10,),
        in_specs=[
            pl.BlockSpec((NC, N_NODES // 10, D), lambda i: (0, i, 0)),
            pl.BlockSpec((1, D), lambda i: (0, 0)),
        ],
        out_specs=pl.BlockSpec((N_NODES // 10, D), lambda i: (i, 0)),
        out_shape=jax.ShapeDtypeStruct((N_NODES, D), jnp.float32),
    )(partials, b2)
    return out


# speedup vs baseline: 1.2078x; 1.2078x over previous
import dataclasses
import functools

import jax
import jax.numpy as jnp
from jax import lax
from jax.experimental import pallas as pl
from jax.experimental.pallas import tpu as pltpu
from jax.experimental.pallas import tpu_sc as plsc

N_NODES = 10000
N_EDGES = 320000
D = 128

NC = 2
NS = 16
L = 16

CHUNK = 128
CPW0 = 120
CPW1 = 40
SPC = 40
MAX_STAGES = max(CPW0, CPW1) // SPC
N_CHUNKS = NS * (CPW0 + CPW1)
E_PAD = N_CHUNKS * CHUNK
RBLK = 80
N_RBLK = N_NODES // RBLK
RB_T = (N_RBLK + NS - 1) // NS


def _matmul_body(x_ref, w_ref, o_ref):
    o_ref[...] = jnp.dot(x_ref[...], w_ref[...],
                         preferred_element_type=jnp.float32)


def _combine_body(p_ref, b_ref, o_ref):
    o_ref[...] = p_ref[0] + p_ref[1] + b_ref[...]


def _sc_spmm(h, row2, col, val):
    mesh = plsc.VectorSubcoreMesh(core_axis_name="c", subcore_axis_name="s")
    cp = pltpu.CompilerParams()
    if "needs_layout_passes" in pltpu.CompilerParams.__dataclass_fields__:
        cp = dataclasses.replace(cp, needs_layout_passes=False)

    @functools.partial(
        pl.kernel,
        compiler_params=cp,
        out_type=jax.ShapeDtypeStruct((NC, N_NODES, D), jnp.float32),
        mesh=mesh,
        scratch_types=[
            pltpu.VMEM((SPC * CHUNK,), jnp.int32),
            pltpu.VMEM((SPC, CHUNK), jnp.int32),
            pltpu.VMEM((SPC * CHUNK,), jnp.float32),
            pltpu.VMEM((CHUNK, D), jnp.float32),
            pltpu.VMEM((CHUNK, D), jnp.float32),
            pltpu.VMEM_SHARED((N_NODES, D), jnp.float32),
            pltpu.SemaphoreType.DMA,
            pltpu.SemaphoreType.DMA,
        ],
    )
    def spmm_kernel(h_hbm, row_hbm, col_hbm, val_hbm, out_hbm,
                    col_v, row_v, val_v, rows_a, rows_b, acc_sh,
                    sem_a, sem_b):
        cid = lax.axis_index("c")
        sid = lax.axis_index("s")

        @pl.loop(0, RBLK)
        def _(e):
            for k in range(D // L):
                rows_a[e, pl.ds(k * L, L)] = jnp.zeros((L,), jnp.float32)

        @pl.loop(0, RB_T)
        def _(t):
            blk = sid + t * NS

            @pl.when(blk < N_RBLK)
            def _():
                pltpu.sync_copy(rows_a.at[pl.ds(0, RBLK)],
                                acc_sh.at[pl.ds(blk * RBLK, RBLK)])

        plsc.subcore_barrier()

        def start_gather(j, buf, sem):
            pltpu.async_copy(
                h_hbm.at[col_v.at[pl.ds(j * CHUNK, CHUNK)]], buf, sem)

        def wait_gather(j, buf, sem):
            pltpu.make_async_copy(
                h_hbm.at[col_v.at[pl.ds(j * CHUNK, CHUNK)]], buf, sem).wait()

        def scale(buf, j):
            @pl.loop(0, CHUNK // L)
            def _(g):
                base_e = j * CHUNK + g * L
                for e in range(L):
                    bcast = plsc.load_gather(
                        val_v, [jnp.full((L,), base_e + e, jnp.int32)])
                    r = g * L + e
                    for k in range(D // L):
                        sl = pl.ds(k * L, L)
                        buf[r, sl] = buf[r, sl] * bcast

        def scatter(buf, j):
            pltpu.sync_copy(buf, acc_sh.at[row_v.at[j]], add=True)

        cpw = jnp.where(cid == 0, CPW0, CPW1)
        cbase = cid * NS * CPW0 + sid * cpw

        for s in range(MAX_STAGES):

            @pl.when(s * SPC < cpw)
            def _():
                stage_c = cbase + s * SPC
                ebase = stage_c * CHUNK
                pltpu.sync_copy(col_hbm.at[pl.ds(ebase, SPC * CHUNK)], col_v)
                pltpu.sync_copy(val_hbm.at[pl.ds(ebase, SPC * CHUNK)], val_v)
                pltpu.sync_copy(row_hbm.at[pl.ds(stage_c, SPC)], row_v)

                start_gather(0, rows_a, sem_a)

                @pl.loop(0, SPC // 2)
                def _(t):
                    j0 = t * 2
                    start_gather(j0 + 1, rows_b, sem_b)

                    wait_gather(j0, rows_a, sem_a)
                    scale(rows_a, j0)
                    scatter(rows_a, j0)

                    @pl.when(j0 + 2 < SPC)
                    def _():
                        start_gather(j0 + 2, rows_a, sem_a)

                    wait_gather(j0 + 1, rows_b, sem_b)
                    scale(rows_b, j0 + 1)
                    scatter(rows_b, j0 + 1)

        plsc.subcore_barrier()

        @pl.loop(0, RB_T)
        def _(t):
            blk = sid + t * NS

            @pl.when(blk < N_RBLK)
            def _():
                pltpu.sync_copy(
                    acc_sh.at[pl.ds(blk * RBLK, RBLK)],
                    out_hbm.at[cid, pl.ds(blk * RBLK, RBLK)])

    return spmm_kernel(h, row2, col, val)


def kernel(x, edge_index, adj_values, W, b):
    row = edge_index[0].astype(jnp.int32)
    col = edge_index[1].astype(jnp.int32)
    val = adj_values.astype(jnp.float32)

    pad = E_PAD - N_EDGES
    row2 = jnp.pad(row, (0, pad)).reshape(N_CHUNKS, CHUNK)
    col = jnp.pad(col, (0, pad))
    val = jnp.pad(val, (0, pad))

    h = pl.pallas_call(
        _matmul_body,
        grid=(10,),
        in_specs=[
            pl.BlockSpec((N_NODES // 10, D), lambda i: (i, 0)),
            pl.BlockSpec((D, D), lambda i: (0, 0)),
        ],
        out_specs=pl.BlockSpec((N_NODES // 10, D), lambda i: (i, 0)),
        out_shape=jax.ShapeDtypeStruct((N_NODES, D), jnp.float32),
    )(x, W)

    partials = _sc_spmm(h, row2, col, val)

    b2 = b.reshape(1, D).astype(jnp.float32)
    out = pl.pallas_call(
        _combine_body,
        grid=(10,),
        in_specs=[
            pl.BlockSpec((NC, N_NODES // 10, D), lambda i: (0, i, 0)),
            pl.BlockSpec((1, D), lambda i: (0, 0)),
        ],
        out_specs=pl.BlockSpec((N_NODES // 10, D), lambda i: (i, 0)),
        out_shape=jax.ShapeDtypeStruct((N_NODES, D), jnp.float32),
    )(partials, b2)
    return out
